# Initial kernel scaffold; baseline (speedup 1.0000x reference)
import jax
import jax.numpy as jnp
from jax.experimental import pallas as pl


def kernel(x, edge_index, batch, w1_0, b1_0, w2_0, b2_0, g_0, be_0, w1_1, b1_1, w2_1, b2_1, g_1, be_1, w1_2, b1_2, w2_2, b2_2, g_2, be_2):
    raise NotImplementedError("write your pallas kernel here")


# speedup vs baseline: 4.3957x; 4.3957x over previous
import functools

import jax
import jax.numpy as jnp
from jax import lax
from jax.experimental import pallas as pl
from jax.experimental.pallas import tpu as pltpu
from jax.experimental.pallas import tpu_sc as plsc

_N = 10000
_E = 320000
_D = 128
_G = 256
_NC = 2
_NS = 16
_NW = _NC * _NS
_NPAD = 10240
_LANES = 16

_EPW = _E // _NW
_ECH = 80
_ENCH = _EPW // _ECH
_RPT = _NPAD // _NS
_ZR = 128

_RW = _NPAD // _NW
_RCH = 64
_RNCH = _RW // _RCH

_mesh = plsc.VectorSubcoreMesh(core_axis_name="c", subcore_axis_name="s")


def _sc_scatter_add_body(x_hbm, src_hbm, dst_hbm, out_hbm,
                         sidx, didx, rows, zbuf, agg_sh, sem):
    cid = lax.axis_index("c")
    sid = lax.axis_index("s")
    wid = sid * _NC + cid

    def _zrow(i, carry):
        for j in range(_D // _LANES):
            zbuf[i, pl.ds(j * _LANES, _LANES)] = jnp.zeros((_LANES,), jnp.float32)
        return carry
    lax.fori_loop(0, _ZR, _zrow, 0)
    for k in range(_RPT // _ZR):
        pltpu.sync_copy(zbuf, agg_sh.at[pl.ds(sid * _RPT + k * _ZR, _ZR)])
    plsc.subcore_barrier()

    base = wid * _EPW

    def _edge_chunk(j, carry):
        off = base + j * _ECH
        pltpu.sync_copy(src_hbm.at[pl.ds(off, _ECH)], sidx)
        pltpu.sync_copy(dst_hbm.at[pl.ds(off, _ECH)], didx)
        pltpu.async_copy(x_hbm.at[sidx], rows, sem).wait()
        pltpu.sync_copy(rows, agg_sh.at[didx], add=True)
        return carry
    lax.fori_loop(0, _ENCH, _edge_chunk, 0)

    plsc.subcore_barrier()
    for k in range(_RPT // _ZR):
        r0 = sid * _RPT + k * _ZR
        pltpu.sync_copy(agg_sh.at[pl.ds(r0, _ZR)], out_hbm.at[cid, pl.ds(r0, _ZR)])


_sc_scatter_add = functools.partial(
    pl.kernel,
    out_type=jax.ShapeDtypeStruct((_NC, _NPAD, _D), jnp.float32),
    mesh=_mesh,
    scratch_types=[
        pltpu.VMEM((_ECH,), jnp.int32),
        pltpu.VMEM((_ECH,), jnp.int32),
        pltpu.VMEM((_ECH, _D), jnp.float32),
        pltpu.VMEM((_ZR, _D), jnp.float32),
        pltpu.VMEM_SHARED((_NPAD, _D), jnp.float32),
        pltpu.SemaphoreType.DMA,
    ],
)(_sc_scatter_add_body)


def _sc_segment_max_body(h_hbm, batch_hbm, out_hbm, bbuf, rowbuf, acc):
    cid = lax.axis_index("c")
    sid = lax.axis_index("s")
    wid = sid * _NC + cid

    ninf = jnp.full((_LANES,), -jnp.inf, jnp.float32)

    def _irow(i, carry):
        for j in range(_D // _LANES):
            acc[i, pl.ds(j * _LANES, _LANES)] = ninf
        return carry
    lax.fori_loop(0, _G, _irow, 0)

    base = wid * _RW
    for c in range(_RNCH):
        pltpu.sync_copy(h_hbm.at[pl.ds(base + c * _RCH, _RCH)], rowbuf)
        pltpu.sync_copy(batch_hbm.at[pl.ds(base + c * _RCH, _RCH)], bbuf)

        def _group(gidx, carry):
            r0 = gidx * _LANES
            bv = bbuf[pl.ds(r0, _LANES)]
            for k in range(_LANES):
                b = bv[k]
                for j in range(_D // _LANES):
                    sl = pl.ds(j * _LANES, _LANES)
                    acc[b, sl] = jnp.maximum(acc[b, sl], rowbuf[r0 + k, sl])
            return carry
        lax.fori_loop(0, _RCH // _LANES, _group, 0)

    pltpu.sync_copy(acc, out_hbm.at[wid])


_sc_segment_max = functools.partial(
    pl.kernel,
    out_type=jax.ShapeDtypeStruct((_NW, _G, _D), jnp.float32),
    mesh=_mesh,
    scratch_types=[
        pltpu.VMEM((_RCH,), jnp.int32),
        pltpu.VMEM((_RCH, _D), jnp.float32),
        pltpu.VMEM((_G, _D), jnp.float32),
    ],
)(_sc_segment_max_body)


def _tc_mlp_bn_body(parts_ref, xin_ref, w1_ref, b1_ref, w2_ref, b2_ref,
                    g_ref, be_ref, out_ref):
    agg = parts_ref[0, 0:_N, :] + parts_ref[1, 0:_N, :] + xin_ref[0:_N, :]
    h = jnp.dot(agg, w1_ref[...], preferred_element_type=jnp.float32)
    h = jnp.maximum(h + b1_ref[...], 0.0)
    h = jnp.dot(h, w2_ref[...], preferred_element_type=jnp.float32)
    h = jnp.maximum(h + b2_ref[...], 0.0)
    mean = jnp.mean(h, axis=0, keepdims=True)
    var = jnp.mean((h - mean) * (h - mean), axis=0, keepdims=True)
    hb = (h - mean) * lax.rsqrt(var + 1e-5) * g_ref[...] + be_ref[...]
    out_ref[0:_N, :] = hb
    out_ref[_N:_NPAD, :] = jnp.full((_NPAD - _N, _D), -jnp.inf, jnp.float32)


_tc_mlp_bn = pl.pallas_call(
    _tc_mlp_bn_body,
    out_shape=jax.ShapeDtypeStruct((_NPAD, _D), jnp.float32),
)


def _tc_combine_body(p0_ref, p1_ref, p2_ref, out_ref):
    out_ref[:, 0 * _D:1 * _D] = jnp.max(p0_ref[...], axis=0)
    out_ref[:, 1 * _D:2 * _D] = jnp.max(p1_ref[...], axis=0)
    out_ref[:, 2 * _D:3 * _D] = jnp.max(p2_ref[...], axis=0)


_tc_combine = pl.pallas_call(
    _tc_combine_body,
    out_shape=jax.ShapeDtypeStruct((_G, 3 * _D), jnp.float32),
)


def kernel(x, edge_index, batch,
           w1_0, b1_0, w2_0, b2_0, g_0, be_0,
           w1_1, b1_1, w2_1, b2_1, g_1, be_1,
           w1_2, b1_2, w2_2, b2_2, g_2, be_2):
    src = edge_index[0]
    dst = edge_index[1]
    batch_pad = jnp.pad(batch, (0, _NPAD - _N))
    h = jnp.pad(x, ((0, _NPAD - _N), (0, 0)))
    params = [(w1_0, b1_0, w2_0, b2_0, g_0, be_0),
              (w1_1, b1_1, w2_1, b2_1, g_1, be_1),
              (w1_2, b1_2, w2_2, b2_2, g_2, be_2)]
    hs = []
    for (w1, b1, w2, b2, g, be) in params:
        parts = _sc_scatter_add(h, src, dst)
        h = _tc_mlp_bn(parts, h, w1, b1.reshape(1, _D), w2, b2.reshape(1, _D),
                       g.reshape(1, _D), be.reshape(1, _D))
        hs.append(h)
    pooled = [_sc_segment_max(hh, batch_pad) for hh in hs]
    return _tc_combine(*pooled)


# speedup vs baseline: 4.9272x; 1.1209x over previous
import functools

import jax
import jax.numpy as jnp
from jax import lax
from jax.experimental import pallas as pl
from jax.experimental.pallas import tpu as pltpu
from jax.experimental.pallas import tpu_sc as plsc

_N = 10000
_E = 320000
_D = 128
_G = 256
_NC = 2
_NS = 16
_NW = _NC * _NS
_NPAD = 10240
_LANES = 16

_ECH = 128
_ENCH = 79
_EPW = _ECH * _ENCH
_EPAD = _NW * _EPW
_RPT = _NPAD // _NS
_ZR = 128

_RW = _NPAD // _NW
_RCH = 64
_RNCH = _RW // _RCH

_mesh = plsc.VectorSubcoreMesh(core_axis_name="c", subcore_axis_name="s")


def _sc_scatter_add_body(x_hbm, packed_hbm, out_hbm,
                         packed_v, sidx, didx, rows, agg_sh, gsem, ssem):
    cid = lax.axis_index("c")
    sid = lax.axis_index("s")
    wid = sid * _NC + cid

    def _zrow(i, carry):
        for j in range(_D // _LANES):
            rows[0, i, pl.ds(j * _LANES, _LANES)] = jnp.zeros((_LANES,),
                                                              jnp.float32)
        return carry
    lax.fori_loop(0, _ECH, _zrow, 0)
    for k in range(_RPT // _ECH):
        pltpu.sync_copy(rows.at[0], agg_sh.at[pl.ds(sid * _RPT + k * _ECH,
                                                    _ECH)])
    plsc.subcore_barrier()

    pltpu.sync_copy(packed_hbm.at[wid], packed_v)

    mask = jnp.full((_LANES,), 16383, jnp.int32)
    shift = jnp.full((_LANES,), 14, jnp.int32)

    def _unpack(j, slot):
        for k in range(_ECH // _LANES):
            sl = pl.ds(k * _LANES, _LANES)
            p = packed_v[j, sl]
            sidx[slot, sl] = jnp.bitwise_and(p, mask)
            didx[slot, sl] = jnp.right_shift(p, shift)

    _unpack(0, 0)
    pltpu.async_copy(x_hbm.at[sidx.at[0]], rows.at[0], gsem)

    def _edge_chunk(j, carry):
        slot = lax.rem(j, 2)
        nslot = lax.rem(j + 1, 2)

        @pl.when(j + 1 < _ENCH)
        def _prefetch():
            @pl.when(j >= 1)
            def _free_slot():
                pltpu.make_async_copy(rows.at[nslot],
                                      agg_sh.at[didx.at[nslot]], ssem).wait()
            _unpack(j + 1, nslot)
            pltpu.async_copy(x_hbm.at[sidx.at[nslot]], rows.at[nslot], gsem)

        pltpu.make_async_copy(x_hbm.at[sidx.at[slot]], rows.at[slot],
                              gsem).wait()
        pltpu.async_copy(rows.at[slot], agg_sh.at[didx.at[slot]], ssem,
                         add=True)
        return carry
    lax.fori_loop(0, _ENCH, _edge_chunk, 0)
    for s in range(2):
        pltpu.make_async_copy(rows.at[s], agg_sh.at[didx.at[s]], ssem).wait()

    plsc.subcore_barrier()
    for k in range(_RPT // _ZR):
        r0 = sid * _RPT + k * _ZR
        pltpu.sync_copy(agg_sh.at[pl.ds(r0, _ZR)], out_hbm.at[cid, pl.ds(r0, _ZR)])


_sc_scatter_add = functools.partial(
    pl.kernel,
    out_type=jax.ShapeDtypeStruct((_NC, _NPAD, _D), jnp.float32),
    mesh=_mesh,
    scratch_types=[
        pltpu.VMEM((_ENCH, _ECH), jnp.int32),
        pltpu.VMEM((2, _ECH), jnp.int32),
        pltpu.VMEM((2, _ECH), jnp.int32),
        pltpu.VMEM((2, _ECH, _D), jnp.float32),
        pltpu.VMEM_SHARED((_NPAD, _D), jnp.float32),
        pltpu.SemaphoreType.DMA,
        pltpu.SemaphoreType.DMA,
    ],
)(_sc_scatter_add_body)


def _sc_segment_max_body(h_hbm, batch_hbm, out_hbm, bbuf, rowbuf, acc):
    cid = lax.axis_index("c")
    sid = lax.axis_index("s")
    wid = sid * _NC + cid

    ninf = jnp.full((_LANES,), -jnp.inf, jnp.float32)

    def _irow(i, carry):
        for j in range(_D // _LANES):
            acc[i, pl.ds(j * _LANES, _LANES)] = ninf
        return carry
    lax.fori_loop(0, _G, _irow, 0)

    base = wid * _RW
    for c in range(_RNCH):
        pltpu.sync_copy(h_hbm.at[pl.ds(base + c * _RCH, _RCH)], rowbuf)
        pltpu.sync_copy(batch_hbm.at[pl.ds(base + c * _RCH, _RCH)], bbuf)

        def _group(gidx, carry):
            r0 = gidx * _LANES
            bv = bbuf[pl.ds(r0, _LANES)]
            for k in range(_LANES):
                b = bv[k]
                for j in range(_D // _LANES):
                    sl = pl.ds(j * _LANES, _LANES)
                    acc[b, sl] = jnp.maximum(acc[b, sl], rowbuf[r0 + k, sl])
            return carry
        lax.fori_loop(0, _RCH // _LANES, _group, 0)

    pltpu.sync_copy(acc, out_hbm.at[wid])


_sc_segment_max = functools.partial(
    pl.kernel,
    out_type=jax.ShapeDtypeStruct((_NW, _G, _D), jnp.float32),
    mesh=_mesh,
    scratch_types=[
        pltpu.VMEM((_RCH,), jnp.int32),
        pltpu.VMEM((_RCH, _D), jnp.float32),
        pltpu.VMEM((_G, _D), jnp.float32),
    ],
)(_sc_segment_max_body)


def _tc_mlp_bn_body(parts_ref, xin_ref, w1_ref, b1_ref, w2_ref, b2_ref,
                    g_ref, be_ref, out_ref):
    agg = parts_ref[0, 0:_N, :] + parts_ref[1, 0:_N, :] + xin_ref[0:_N, :]
    h = jnp.dot(agg, w1_ref[...], preferred_element_type=jnp.float32)
    h = jnp.maximum(h + b1_ref[...], 0.0)
    h = jnp.dot(h, w2_ref[...], preferred_element_type=jnp.float32)
    h = jnp.maximum(h + b2_ref[...], 0.0)
    mean = jnp.mean(h, axis=0, keepdims=True)
    var = jnp.mean((h - mean) * (h - mean), axis=0, keepdims=True)
    hb = (h - mean) * lax.rsqrt(var + 1e-5) * g_ref[...] + be_ref[...]
    out_ref[0:_N, :] = hb
    out_ref[_N:_NPAD, :] = jnp.full((_NPAD - _N, _D), -jnp.inf, jnp.float32)


_tc_mlp_bn = pl.pallas_call(
    _tc_mlp_bn_body,
    out_shape=jax.ShapeDtypeStruct((_NPAD, _D), jnp.float32),
)


def _tc_combine_body(p0_ref, p1_ref, p2_ref, out_ref):
    out_ref[:, 0 * _D:1 * _D] = jnp.max(p0_ref[...], axis=0)
    out_ref[:, 1 * _D:2 * _D] = jnp.max(p1_ref[...], axis=0)
    out_ref[:, 2 * _D:3 * _D] = jnp.max(p2_ref[...], axis=0)


_tc_combine = pl.pallas_call(
    _tc_combine_body,
    out_shape=jax.ShapeDtypeStruct((_G, 3 * _D), jnp.float32),
)


def kernel(x, edge_index, batch,
           w1_0, b1_0, w2_0, b2_0, g_0, be_0,
           w1_1, b1_1, w2_1, b2_1, g_1, be_1,
           w1_2, b1_2, w2_2, b2_2, g_2, be_2):
    packed = jnp.concatenate(
        [edge_index[0] | (edge_index[1] << 14),
         jnp.full((_EPAD - _E,), _N << 14, jnp.int32)]
    ).reshape(_NW, _ENCH, _ECH)
    batch_pad = jnp.pad(batch, (0, _NPAD - _N))
    h = jnp.pad(x, ((0, _NPAD - _N), (0, 0)))
    params = [(w1_0, b1_0, w2_0, b2_0, g_0, be_0),
              (w1_1, b1_1, w2_1, b2_1, g_1, be_1),
              (w1_2, b1_2, w2_2, b2_2, g_2, be_2)]
    hs = []
    for (w1, b1, w2, b2, g, be) in params:
        parts = _sc_scatter_add(h, packed)
        h = _tc_mlp_bn(parts, h, w1, b1.reshape(1, _D), w2, b2.reshape(1, _D),
                       g.reshape(1, _D), be.reshape(1, _D))
        hs.append(h)
    pooled = [_sc_segment_max(hh, batch_pad) for hh in hs]
    return _tc_combine(*pooled)


# speedup vs baseline: 7.8883x; 1.6010x over previous
import functools

import jax
import jax.numpy as jnp
from jax import lax
from jax.experimental import pallas as pl
from jax.experimental.pallas import tpu as pltpu
from jax.experimental.pallas import tpu_sc as plsc

_N = 10000
_E = 320000
_D = 128
_G = 256
_NC = 2
_NS = 16
_NW = _NC * _NS
_NPAD = 10240
_LANES = 16

_ECH = 128
_ENCH0 = 110
_ENCH1 = 47
_ENCHMAX = max(_ENCH0, _ENCH1)
_EPAD = _NS * (_ENCH0 + _ENCH1) * _ECH
_RPT = _NPAD // _NS
_ZR = 128

_RW = _NPAD // _NW
_RCH = 64
_RNCH = _RW // _RCH

_mesh = plsc.VectorSubcoreMesh(core_axis_name="c", subcore_axis_name="s")


def _sc_scatter_add_body(x_hbm, packed_hbm, out_hbm,
                         packed_v, sidx, didx, rows, agg_sh, gsem, ssem):
    cid = lax.axis_index("c")
    sid = lax.axis_index("s")
    wid = sid * _NC + cid

    def _zrow(i, carry):
        for j in range(_D // _LANES):
            rows[0, i, pl.ds(j * _LANES, _LANES)] = jnp.zeros((_LANES,),
                                                              jnp.float32)
        return carry
    lax.fori_loop(0, _ECH, _zrow, 0)
    for k in range(_RPT // _ECH):
        pltpu.sync_copy(rows.at[0], agg_sh.at[pl.ds(sid * _RPT + k * _ECH,
                                                    _ECH)])
    plsc.subcore_barrier()

    pltpu.sync_copy(packed_hbm.at[wid], packed_v)
    nch = jnp.where(cid == 0, _ENCH0, _ENCH1)

    mask = jnp.full((_LANES,), 16383, jnp.int32)
    shift = jnp.full((_LANES,), 14, jnp.int32)

    def _unpack(j, slot):
        for k in range(_ECH // _LANES):
            sl = pl.ds(k * _LANES, _LANES)
            p = packed_v[j, sl]
            sidx[slot, sl] = jnp.bitwise_and(p, mask)
            didx[slot, sl] = jnp.right_shift(p, shift)

    _unpack(0, 0)
    pltpu.async_copy(x_hbm.at[sidx.at[0]], rows.at[0], gsem)

    def _edge_chunk(j, carry):
        slot = lax.rem(j, 2)
        nslot = lax.rem(j + 1, 2)

        @pl.when(j + 1 < nch)
        def _prefetch():
            @pl.when(j >= 1)
            def _free_slot():
                pltpu.make_async_copy(rows.at[nslot],
                                      agg_sh.at[didx.at[nslot]], ssem).wait()
            _unpack(j + 1, nslot)
            pltpu.async_copy(x_hbm.at[sidx.at[nslot]], rows.at[nslot], gsem)

        pltpu.make_async_copy(x_hbm.at[sidx.at[slot]], rows.at[slot],
                              gsem).wait()
        pltpu.async_copy(rows.at[slot], agg_sh.at[didx.at[slot]], ssem,
                         add=True)
        return carry
    lax.fori_loop(0, nch, _edge_chunk, 0)
    for s in range(2):
        pltpu.make_async_copy(rows.at[s], agg_sh.at[didx.at[s]], ssem).wait()

    plsc.subcore_barrier()
    for k in range(_RPT // _ZR):
        r0 = sid * _RPT + k * _ZR
        pltpu.sync_copy(agg_sh.at[pl.ds(r0, _ZR)], out_hbm.at[cid, pl.ds(r0, _ZR)])


_sc_scatter_add = functools.partial(
    pl.kernel,
    out_type=jax.ShapeDtypeStruct((_NC, _NPAD, _D), jnp.float32),
    mesh=_mesh,
    scratch_types=[
        pltpu.VMEM((_ENCHMAX, _ECH), jnp.int32),
        pltpu.VMEM((2, _ECH), jnp.int32),
        pltpu.VMEM((2, _ECH), jnp.int32),
        pltpu.VMEM((2, _ECH, _D), jnp.float32),
        pltpu.VMEM_SHARED((_NPAD, _D), jnp.float32),
        pltpu.SemaphoreType.DMA,
        pltpu.SemaphoreType.DMA,
    ],
)(_sc_scatter_add_body)


def _sc_segment_max_body(h_hbm, batch_hbm, out_hbm, bbuf, rowbuf, acc):
    cid = lax.axis_index("c")
    sid = lax.axis_index("s")
    wid = sid * _NC + cid

    ninf = jnp.full((_LANES,), -jnp.inf, jnp.float32)

    def _irow(i, carry):
        for j in range(_D // _LANES):
            acc[i, pl.ds(j * _LANES, _LANES)] = ninf
        return carry
    lax.fori_loop(0, _G, _irow, 0)

    base = wid * _RW
    for c in range(_RNCH):
        pltpu.sync_copy(h_hbm.at[pl.ds(base + c * _RCH, _RCH)], rowbuf)
        pltpu.sync_copy(batch_hbm.at[pl.ds(base + c * _RCH, _RCH)], bbuf)

        def _group(gidx, carry):
            r0 = gidx * _LANES
            bv = bbuf[pl.ds(r0, _LANES)]
            for k in range(_LANES):
                b = bv[k]
                for j in range(_D // _LANES):
                    sl = pl.ds(j * _LANES, _LANES)
                    acc[b, sl] = jnp.maximum(acc[b, sl], rowbuf[r0 + k, sl])
            return carry
        lax.fori_loop(0, _RCH // _LANES, _group, 0)

    pltpu.sync_copy(acc, out_hbm.at[wid])


_sc_segment_max = functools.partial(
    pl.kernel,
    out_type=jax.ShapeDtypeStruct((_NW, _G, _D), jnp.float32),
    mesh=_mesh,
    scratch_types=[
        pltpu.VMEM((_RCH,), jnp.int32),
        pltpu.VMEM((_RCH, _D), jnp.float32),
        pltpu.VMEM((_G, _D), jnp.float32),
    ],
)(_sc_segment_max_body)


def _tc_mlp_bn_body(parts_ref, xin_ref, w1_ref, b1_ref, w2_ref, b2_ref,
                    g_ref, be_ref, out_ref):
    agg = parts_ref[0, 0:_N, :] + parts_ref[1, 0:_N, :] + xin_ref[0:_N, :]
    h = jnp.dot(agg, w1_ref[...], preferred_element_type=jnp.float32)
    h = jnp.maximum(h + b1_ref[...], 0.0)
    h = jnp.dot(h, w2_ref[...], preferred_element_type=jnp.float32)
    h = jnp.maximum(h + b2_ref[...], 0.0)
    mean = jnp.mean(h, axis=0, keepdims=True)
    var = jnp.mean((h - mean) * (h - mean), axis=0, keepdims=True)
    hb = (h - mean) * lax.rsqrt(var + 1e-5) * g_ref[...] + be_ref[...]
    out_ref[0:_N, :] = hb
    out_ref[_N:_NPAD, :] = jnp.full((_NPAD - _N, _D), -jnp.inf, jnp.float32)


_tc_mlp_bn = pl.pallas_call(
    _tc_mlp_bn_body,
    out_shape=jax.ShapeDtypeStruct((_NPAD, _D), jnp.float32),
)


def _tc_combine_body(p0_ref, p1_ref, p2_ref, out_ref):
    out_ref[:, 0 * _D:1 * _D] = jnp.max(p0_ref[...], axis=0)
    out_ref[:, 1 * _D:2 * _D] = jnp.max(p1_ref[...], axis=0)
    out_ref[:, 2 * _D:3 * _D] = jnp.max(p2_ref[...], axis=0)


_tc_combine = pl.pallas_call(
    _tc_combine_body,
    out_shape=jax.ShapeDtypeStruct((_G, 3 * _D), jnp.float32),
)


def kernel(x, edge_index, batch,
           w1_0, b1_0, w2_0, b2_0, g_0, be_0,
           w1_1, b1_1, w2_1, b2_1, g_1, be_1,
           w1_2, b1_2, w2_2, b2_2, g_2, be_2):
    flat = jnp.concatenate(
        [edge_index[0] | (edge_index[1] << 14),
         jnp.full((_EPAD - _E,), _N << 14, jnp.int32)]
    )
    rows_per_wid = []
    off = 0
    for sid in range(_NS):
        for cid in range(_NC):
            cnt = (_ENCH0 if cid == 0 else _ENCH1) * _ECH
            row = flat[off:off + cnt]
            off += cnt
            if cnt < _ENCHMAX * _ECH:
                row = jnp.concatenate(
                    [row, jnp.full((_ENCHMAX * _ECH - cnt,), _N << 14,
                                   jnp.int32)])
            rows_per_wid.append(row)
    packed = jnp.stack(rows_per_wid).reshape(_NW, _ENCHMAX, _ECH)
    batch_pad = jnp.pad(batch, (0, _NPAD - _N))
    h = jnp.pad(x, ((0, _NPAD - _N), (0, 0)))
    params = [(w1_0, b1_0, w2_0, b2_0, g_0, be_0),
              (w1_1, b1_1, w2_1, b2_1, g_1, be_1),
              (w1_2, b1_2, w2_2, b2_2, g_2, be_2)]
    hs = []
    for (w1, b1, w2, b2, g, be) in params:
        parts = _sc_scatter_add(h, packed)
        h = _tc_mlp_bn(parts, h, w1, b1.reshape(1, _D), w2, b2.reshape(1, _D),
                       g.reshape(1, _D), be.reshape(1, _D))
        hs.append(h)
    pooled = [_sc_segment_max(hh, batch_pad) for hh in hs]
    return _tc_combine(*pooled)
